# Initial kernel scaffold; baseline (speedup 1.0000x reference)
import jax
import jax.numpy as jnp
from jax.experimental import pallas as pl


def kernel(h, domain, emb):
    raise NotImplementedError("write your pallas kernel here")


# speedup vs baseline: 1.7732x; 1.7732x over previous
import functools

import jax
import jax.numpy as jnp
from jax import lax
from jax.experimental import pallas as pl
from jax.experimental.pallas import tpu as pltpu
from jax.experimental.pallas import tpu_sc as plsc

_B = 16384
_D = 128
_LANES = 16
_NC = 2
_NS = 16
_NW = _NC * _NS
_BPW = _B // _NW
_CHUNK = 128
_NCH = _BPW // _CHUNK


def _body(h_hbm, dom_hbm, emb_hbm, out_hbm, idx_v, hbuf, ebuf, sem_h, sem_e):
    wid = lax.axis_index("s") * _NC + lax.axis_index("c")
    base = wid * _BPW

    def chunk(ci, carry):
        off = base + ci * _CHUNK
        pltpu.sync_copy(dom_hbm.at[pl.ds(off, _CHUNK)], idx_v)
        h_cp = pltpu.make_async_copy(h_hbm.at[pl.ds(off, _CHUNK)], hbuf, sem_h)
        h_cp.start()
        e_cp = pltpu.make_async_copy(emb_hbm.at[idx_v], ebuf, sem_e)
        e_cp.start()
        h_cp.wait()
        e_cp.wait()

        def row(r, c2):
            for c in range(_D // _LANES):
                sl = pl.ds(c * _LANES, _LANES)
                plsc.addupdate(hbuf.at[r, sl], ebuf[r, sl])
            return c2

        lax.fori_loop(0, _CHUNK, row, 0)
        pltpu.sync_copy(hbuf, out_hbm.at[pl.ds(off, _CHUNK)])
        return carry

    lax.fori_loop(0, _NCH, chunk, 0)


@jax.jit
def _domain_token(h, domain, emb):
    mesh = plsc.VectorSubcoreMesh(core_axis_name="c", subcore_axis_name="s")
    return pl.kernel(
        _body,
        out_type=jax.ShapeDtypeStruct((_B, _D), jnp.float32),
        mesh=mesh,
        scratch_types=[
            pltpu.VMEM((_CHUNK,), jnp.int32),
            pltpu.VMEM((_CHUNK, _D), jnp.float32),
            pltpu.VMEM((_CHUNK, _D), jnp.float32),
            pltpu.SemaphoreType.DMA,
            pltpu.SemaphoreType.DMA,
        ],
    )(h, domain, emb)


def kernel(h, domain, emb):
    return _domain_token(h, domain.astype(jnp.int32), emb)


# speedup vs baseline: 1.8239x; 1.0286x over previous
import functools

import jax
import jax.numpy as jnp
from jax import lax
from jax.experimental import pallas as pl
from jax.experimental.pallas import tpu as pltpu
from jax.experimental.pallas import tpu_sc as plsc

_B = 16384
_D = 128
_LANES = 16
_NC = 2
_NS = 16
_NW = _NC * _NS
_BPW = _B // _NW
_CHUNK = 128
_NCH = _BPW // _CHUNK


def _body(h_hbm, dom_hbm, emb_hbm, out_hbm, idx_v, hbuf, ebuf, sem_h, sem_e):
    wid = lax.axis_index("s") * _NC + lax.axis_index("c")
    base = wid * _BPW

    def chunk(ci, carry):
        off = base + ci * _CHUNK
        pltpu.sync_copy(dom_hbm.at[pl.ds(off, _CHUNK)], idx_v)
        h_cp = pltpu.make_async_copy(h_hbm.at[pl.ds(off, _CHUNK)], hbuf, sem_h)
        h_cp.start()
        h_cp.wait()
        pltpu.async_copy(emb_hbm.at[idx_v], hbuf, sem_e, add=True).wait()
        pltpu.sync_copy(hbuf, out_hbm.at[pl.ds(off, _CHUNK)])
        return carry

    lax.fori_loop(0, _NCH, chunk, 0)


@jax.jit
def _domain_token(h, domain, emb):
    mesh = plsc.VectorSubcoreMesh(core_axis_name="c", subcore_axis_name="s")
    return pl.kernel(
        _body,
        out_type=jax.ShapeDtypeStruct((_B, _D), jnp.float32),
        mesh=mesh,
        scratch_types=[
            pltpu.VMEM((_CHUNK,), jnp.int32),
            pltpu.VMEM((_CHUNK, _D), jnp.float32),
            pltpu.VMEM((_CHUNK, _D), jnp.float32),
            pltpu.SemaphoreType.DMA,
            pltpu.SemaphoreType.DMA,
        ],
    )(h, domain, emb)


def kernel(h, domain, emb):
    return _domain_token(h, domain.astype(jnp.int32), emb)


# speedup vs baseline: 2.0903x; 1.1461x over previous
import functools

import jax
import jax.numpy as jnp
from jax import lax
from jax.experimental import pallas as pl
from jax.experimental.pallas import tpu as pltpu
from jax.experimental.pallas import tpu_sc as plsc

_B = 16384
_D = 128
_NC = 2
_NS = 16
_NW = _NC * _NS
_BPW = _B // _NW
_CHUNK = 128
_NCH = _BPW // _CHUNK


def _body(h_hbm, dom_hbm, emb_hbm, out_hbm,
          idx_v, hbufs, sem_i, sem_h, sem_e, sem_o):
    wid = lax.axis_index("s") * _NC + lax.axis_index("c")
    base = wid * _BPW

    for ci in range(_NCH):
        off = base + ci * _CHUNK
        pltpu.make_async_copy(
            dom_hbm.at[pl.ds(off, _CHUNK)], idx_v.at[ci], sem_i.at[ci]).start()
        pltpu.make_async_copy(
            h_hbm.at[pl.ds(off, _CHUNK)], hbufs[ci], sem_h.at[ci]).start()

    gadds = []
    for ci in range(_NCH):
        pltpu.make_async_copy(
            dom_hbm.at[pl.ds(base + ci * _CHUNK, _CHUNK)],
            idx_v.at[ci], sem_i.at[ci]).wait()
        pltpu.make_async_copy(
            h_hbm.at[pl.ds(base + ci * _CHUNK, _CHUNK)],
            hbufs[ci], sem_h.at[ci]).wait()
        cp = pltpu.async_copy(
            emb_hbm.at[idx_v.at[ci]], hbufs[ci], sem_e.at[ci], add=True)
        gadds.append(cp)

    wbs = []
    for ci in range(_NCH):
        gadds[ci].wait()
        cp = pltpu.make_async_copy(
            hbufs[ci], out_hbm.at[pl.ds(base + ci * _CHUNK, _CHUNK)],
            sem_o.at[ci])
        cp.start()
        wbs.append(cp)

    for ci in range(_NCH):
        wbs[ci].wait()


@jax.jit
def _domain_token(h, domain, emb):
    mesh = plsc.VectorSubcoreMesh(core_axis_name="c", subcore_axis_name="s")
    return pl.kernel(
        _body,
        out_type=jax.ShapeDtypeStruct((_B, _D), jnp.float32),
        mesh=mesh,
        scratch_types=[
            pltpu.VMEM((_NCH, _CHUNK), jnp.int32),
            [pltpu.VMEM((_CHUNK, _D), jnp.float32) for _ in range(_NCH)],
            pltpu.SemaphoreType.DMA((_NCH,)),
            pltpu.SemaphoreType.DMA((_NCH,)),
            pltpu.SemaphoreType.DMA((_NCH,)),
            pltpu.SemaphoreType.DMA((_NCH,)),
        ],
    )(h, domain, emb)


def kernel(h, domain, emb):
    return _domain_token(h, domain.astype(jnp.int32), emb)
